# Initial kernel scaffold; baseline (speedup 1.0000x reference)
import jax
import jax.numpy as jnp
from jax.experimental import pallas as pl


def kernel(positions, atomic_numbers, batch, global_features, params):
    raise NotImplementedError("write your pallas kernel here")


# speedup vs baseline: 4.2629x; 4.2629x over previous
import jax
import jax.numpy as jnp
from jax.experimental import pallas as pl

_N = 4096
_G = 64
_DN = 128
_DE = 128
_DG = 64
_CUTOFF = 0.098
_STEPS = 3
_EC = 131072

_BR1 = 256
_BE = 4096
_BR4 = 512


def _collapse(layers, dense):
    W = layers[0]["W"]
    b = layers[0]["b"]
    for lp in list(layers[1:]) + [dense]:
        b = b @ lp["W"] + lp["b"]
        W = W @ lp["W"]
    return W, b


def _k1_body(mask_ref, f_ref, out_ref):
    m = mask_ref[...].astype(jnp.float32)
    out_ref[...] = jnp.dot(m, f_ref[...], preferred_element_type=jnp.float32)


def _pair_sums(mask_i8, F):
    return pl.pallas_call(
        _k1_body,
        grid=(_N // _BR1,),
        in_specs=[
            pl.BlockSpec((_BR1, _N), lambda i: (i, 0)),
            pl.BlockSpec((_N, 256), lambda i: (0, 0)),
        ],
        out_specs=pl.BlockSpec((_BR1, 256), lambda i: (i, 0)),
        out_shape=jax.ShapeDtypeStruct((_N, 256), jnp.float32),
    )(mask_i8, F)


def _k2_body(coeff, dist_ref, ev_ref, nasum_ref, gterm_ref, off_ref, cpi_ref,
             amask_ref, xe_ref):
    d = dist_ref[...]
    a = jnp.exp(coeff * (d - off_ref[...]) ** 2)
    amask_ref[...] = a * ev_ref[...]
    xe_ref[...] = (0.5 * nasum_ref[...]
                   + jnp.dot(a, cpi_ref[...], preferred_element_type=jnp.float32)
                   + gterm_ref[...])


def _edge_kernel(coeff, dist, ev_f, nasum, gterm, off_row, cpi):
    import functools
    body = functools.partial(_k2_body, coeff)
    return pl.pallas_call(
        body,
        grid=(_EC // _BE,),
        in_specs=[
            pl.BlockSpec((_BE, 1), lambda i: (i, 0)),
            pl.BlockSpec((_BE, 1), lambda i: (i, 0)),
            pl.BlockSpec((_BE, _DE), lambda i: (i, 0)),
            pl.BlockSpec((_BE, _DE), lambda i: (i, 0)),
            pl.BlockSpec((1, _DE), lambda i: (0, 0)),
            pl.BlockSpec((_DE, _DE), lambda i: (0, 0)),
        ],
        out_specs=[
            pl.BlockSpec((_BE, _DE), lambda i: (i, 0)),
            pl.BlockSpec((_BE, _DE), lambda i: (i, 0)),
        ],
        out_shape=[
            jax.ShapeDtypeStruct((_EC, _DE), jnp.float32),
            jax.ShapeDtypeStruct((_EC, _DE), jnp.float32),
        ],
    )(dist, ev_f, nasum, gterm, off_row, cpi)


def _k4_body(node_ref, nm_ref, am_ref, deg_ref, gb_ref, we_ref, wn_ref,
             be_ref, bn_ref, nupd_ref, nnew_ref, p_ref):
    node = node_ref[...]
    deg = deg_ref[...]
    invd = 1.0 / jnp.maximum(deg, 1.0)
    has = (deg > 0.0).astype(jnp.float32)
    A = we_ref[0:128, :]
    B = we_ref[128:256, :]
    C = we_ref[256:384, :]
    D = we_ref[384:448, :]
    f32 = jnp.float32
    nA = jnp.dot(node, A, preferred_element_type=f32)
    nmB = jnp.dot(nm_ref[...], B, preferred_element_type=f32)
    amC = jnp.dot(am_ref[...], C, preferred_element_type=f32)
    gD = jnp.dot(gb_ref[...], D, preferred_element_type=f32)
    be = be_ref[...]
    emean = has * (nA + invd * (nmB + amC) + gD + be)
    p_ref[...] = deg * nA + nmB + amC + deg * be
    WnN = wn_ref[0:128, :]
    WnE = wn_ref[128:256, :]
    WnG = wn_ref[256:320, :]
    nupd = (jnp.dot(node, WnN, preferred_element_type=f32)
            + jnp.dot(emean, WnE, preferred_element_type=f32)
            + jnp.dot(gb_ref[...], WnG, preferred_element_type=f32)
            + bn_ref[...])
    nupd_ref[...] = nupd
    nnew_ref[...] = nupd + node


def _node_kernel(node, nm, am, deg, gb, We, Wn, be_row, bn_row):
    return pl.pallas_call(
        _k4_body,
        grid=(_N // _BR4,),
        in_specs=[
            pl.BlockSpec((_BR4, _DN), lambda i: (i, 0)),
            pl.BlockSpec((_BR4, _DN), lambda i: (i, 0)),
            pl.BlockSpec((_BR4, _DE), lambda i: (i, 0)),
            pl.BlockSpec((_BR4, 1), lambda i: (i, 0)),
            pl.BlockSpec((_BR4, _DG), lambda i: (i, 0)),
            pl.BlockSpec((448, 128), lambda i: (0, 0)),
            pl.BlockSpec((320, 128), lambda i: (0, 0)),
            pl.BlockSpec((1, 128), lambda i: (0, 0)),
            pl.BlockSpec((1, 128), lambda i: (0, 0)),
        ],
        out_specs=[
            pl.BlockSpec((_BR4, _DN), lambda i: (i, 0)),
            pl.BlockSpec((_BR4, _DN), lambda i: (i, 0)),
            pl.BlockSpec((_BR4, _DE), lambda i: (i, 0)),
        ],
        out_shape=[
            jax.ShapeDtypeStruct((_N, _DN), jnp.float32),
            jax.ShapeDtypeStruct((_N, _DN), jnp.float32),
            jax.ShapeDtypeStruct((_N, _DE), jnp.float32),
        ],
    )(node, nm, am, deg, gb, We, Wn, be_row, bn_row)


def _s2s(p, x, ids, num, w):
    d = x.shape[1]
    h = jnp.zeros((num, d), x.dtype)
    c = jnp.zeros((num, d), x.dtype)
    q_star = jnp.zeros((num, 2 * d), x.dtype)
    for _ in range(_STEPS):
        gates = q_star @ p["W_ih"] + p["b_ih"] + h @ p["W_hh"] + p["b_hh"]
        i, f, g, o = jnp.split(gates, 4, axis=1)
        c = jax.nn.sigmoid(f) * c + jax.nn.sigmoid(i) * jnp.tanh(g)
        h = jax.nn.sigmoid(o) * jnp.tanh(c)
        q = h
        e = jnp.sum(x * q[ids], axis=-1)
        e = jnp.where(w, e, -jnp.inf)
        m = jax.ops.segment_max(e, ids, num_segments=num)
        a = jnp.where(w, jnp.exp(e - m[ids]), 0.0)
        s = jax.ops.segment_sum(a, ids, num_segments=num)
        a = jnp.where(w, a / s[ids], 0.0)
        r = jax.ops.segment_sum(a[:, None] * x, ids, num_segments=num)
        q_star = jnp.concatenate([q, r], axis=1)
    return q_star


def kernel(positions, atomic_numbers, batch, global_features, params):
    gn = params["gn"]
    We, be = _collapse(gn["edge"], gn["edge_dense"])
    Wn, bn = _collapse(gn["node"], gn["node_dense"])
    Wg, bg = _collapse(gn["glob"], gn["global_dense"])
    Wf, bf = _collapse([params["dense1"], params["dense2"]], params["out"])

    sq = jnp.sum(positions * positions, axis=1)
    d2 = sq[:, None] + sq[None, :] - 2.0 * (positions @ positions.T)
    mask = (d2 < _CUTOFF * _CUTOFF) & (~jnp.eye(_N, dtype=bool))
    src, dst = jnp.nonzero(mask, size=_EC, fill_value=0)
    ev = mask[src, dst]
    src = src.astype(jnp.int32)
    dst = dst.astype(jnp.int32)

    node = params["embedding"][atomic_numbers]
    glob = global_features
    gb = glob[batch]

    F = jnp.concatenate(
        [node, jnp.ones((_N, 1), jnp.float32), jnp.zeros((_N, 127), jnp.float32)],
        axis=1)
    ps = _pair_sums(mask.astype(jnp.int8), F)
    nm = ps[:, :_DN]
    deg = ps[:, _DN:_DN + 1]

    offset = jnp.linspace(0.0, _CUTOFF, _DE)
    step = _CUTOFF / (_DE - 1)
    coeff = -0.5 / (step * step)
    A = We[:128]
    B = We[128:256]
    CpI = We[256:384] + jnp.eye(_DE, dtype=jnp.float32)
    D = We[384:]
    na = node @ (A + B)
    gDbe = glob @ D + be
    dist = jnp.linalg.norm(positions[src] - positions[dst], axis=-1)
    nasum = na[src] + na[dst]
    gterm = 0.5 * (gDbe[batch[src]] + gDbe[batch[dst]])
    amask, xe = _edge_kernel(coeff, dist[:, None], ev.astype(jnp.float32)[:, None],
                             nasum, gterm, offset[None, :], CpI)

    am = jax.ops.segment_sum(amask, src, num_segments=_N)

    nupd, nnew, P = _node_kernel(node, nm, am, deg, gb, We, Wn,
                                 be[None, :], bn[None, :])

    S = jax.ops.segment_sum(P, batch, num_segments=_G)
    Dg = jax.ops.segment_sum(deg[:, 0], batch, num_segments=_G)
    has_e = (Dg > 0.0).astype(jnp.float32)[:, None]
    egmean = S / jnp.maximum(Dg, 1.0)[:, None] + has_e * (glob @ D)
    nsum = jax.ops.segment_sum(nupd, batch, num_segments=_G)
    cnt = jax.ops.segment_sum(jnp.ones((_N,), jnp.float32), batch, num_segments=_G)
    ngmean = nsum / jnp.maximum(cnt, 1.0)[:, None]
    gin = jnp.concatenate([egmean, ngmean, glob], axis=1)
    gnew = gin @ Wg + bg + glob

    node_r = _s2s(params["s2s_nodes"], nnew, batch, _G,
                  jnp.ones((_N,), bool))
    edge_r = _s2s(params["s2s_edges"], xe, batch[src], _G, ev)
    y = jnp.concatenate([node_r, edge_r, gnew], axis=1) @ Wf + bf
    return y


# speedup vs baseline: 5.8580x; 1.3742x over previous
import jax
import jax.numpy as jnp
from jax.experimental import pallas as pl

_N = 4096
_G = 64
_DN = 128
_DE = 128
_DG = 64
_CUTOFF = 0.098
_STEPS = 3
_EC = 131072

_BR1 = 256
_BE = 4096
_BR4 = 512


def _collapse(layers, dense):
    W = layers[0]["W"]
    b = layers[0]["b"]
    for lp in list(layers[1:]) + [dense]:
        b = b @ lp["W"] + lp["b"]
        W = W @ lp["W"]
    return W, b


def _k1_body(mask_ref, f_ref, out_ref):
    m = mask_ref[...].astype(jnp.float32)
    out_ref[...] = jnp.dot(m, f_ref[...], preferred_element_type=jnp.float32)


def _pair_sums(mask_i8, F):
    return pl.pallas_call(
        _k1_body,
        grid=(_N // _BR1,),
        in_specs=[
            pl.BlockSpec((_BR1, _N), lambda i: (i, 0)),
            pl.BlockSpec((_N, 256), lambda i: (0, 0)),
        ],
        out_specs=pl.BlockSpec((_BR1, 256), lambda i: (i, 0)),
        out_shape=jax.ShapeDtypeStruct((_N, 256), jnp.float32),
    )(mask_i8, F)


def _k2_body(coeff, dist_ref, ev_ref, nasum_ref, gterm_ref, off_ref, cpi_ref,
             amask_ref, xe_ref):
    d = dist_ref[...]
    a = jnp.exp(coeff * (d - off_ref[...]) ** 2)
    amask_ref[...] = a * ev_ref[...]
    xe_ref[...] = (0.5 * nasum_ref[...]
                   + jnp.dot(a, cpi_ref[...], preferred_element_type=jnp.float32)
                   + gterm_ref[...])


def _edge_kernel(coeff, dist, ev_f, nasum, gterm, off_row, cpi):
    import functools
    body = functools.partial(_k2_body, coeff)
    return pl.pallas_call(
        body,
        grid=(_EC // _BE,),
        in_specs=[
            pl.BlockSpec((_BE, 1), lambda i: (i, 0)),
            pl.BlockSpec((_BE, 1), lambda i: (i, 0)),
            pl.BlockSpec((_BE, _DE), lambda i: (i, 0)),
            pl.BlockSpec((_BE, _DE), lambda i: (i, 0)),
            pl.BlockSpec((1, _DE), lambda i: (0, 0)),
            pl.BlockSpec((_DE, _DE), lambda i: (0, 0)),
        ],
        out_specs=[
            pl.BlockSpec((_BE, _DE), lambda i: (i, 0)),
            pl.BlockSpec((_BE, _DE), lambda i: (i, 0)),
        ],
        out_shape=[
            jax.ShapeDtypeStruct((_EC, _DE), jnp.float32),
            jax.ShapeDtypeStruct((_EC, _DE), jnp.float32),
        ],
    )(dist, ev_f, nasum, gterm, off_row, cpi)


def _k4_body(node_ref, nm_ref, am_ref, deg_ref, gb_ref, we_ref, wn_ref,
             be_ref, bn_ref, nupd_ref, nnew_ref, p_ref):
    node = node_ref[...]
    deg = deg_ref[...]
    invd = 1.0 / jnp.maximum(deg, 1.0)
    has = (deg > 0.0).astype(jnp.float32)
    A = we_ref[0:128, :]
    B = we_ref[128:256, :]
    C = we_ref[256:384, :]
    D = we_ref[384:448, :]
    f32 = jnp.float32
    nA = jnp.dot(node, A, preferred_element_type=f32)
    nmB = jnp.dot(nm_ref[...], B, preferred_element_type=f32)
    amC = jnp.dot(am_ref[...], C, preferred_element_type=f32)
    gD = jnp.dot(gb_ref[...], D, preferred_element_type=f32)
    be = be_ref[...]
    emean = has * (nA + invd * (nmB + amC) + gD + be)
    p_ref[...] = deg * nA + nmB + amC + deg * be
    WnN = wn_ref[0:128, :]
    WnE = wn_ref[128:256, :]
    WnG = wn_ref[256:320, :]
    nupd = (jnp.dot(node, WnN, preferred_element_type=f32)
            + jnp.dot(emean, WnE, preferred_element_type=f32)
            + jnp.dot(gb_ref[...], WnG, preferred_element_type=f32)
            + bn_ref[...])
    nupd_ref[...] = nupd
    nnew_ref[...] = nupd + node


def _node_kernel(node, nm, am, deg, gb, We, Wn, be_row, bn_row):
    return pl.pallas_call(
        _k4_body,
        grid=(_N // _BR4,),
        in_specs=[
            pl.BlockSpec((_BR4, _DN), lambda i: (i, 0)),
            pl.BlockSpec((_BR4, _DN), lambda i: (i, 0)),
            pl.BlockSpec((_BR4, _DE), lambda i: (i, 0)),
            pl.BlockSpec((_BR4, 1), lambda i: (i, 0)),
            pl.BlockSpec((_BR4, _DG), lambda i: (i, 0)),
            pl.BlockSpec((448, 128), lambda i: (0, 0)),
            pl.BlockSpec((320, 128), lambda i: (0, 0)),
            pl.BlockSpec((1, 128), lambda i: (0, 0)),
            pl.BlockSpec((1, 128), lambda i: (0, 0)),
        ],
        out_specs=[
            pl.BlockSpec((_BR4, _DN), lambda i: (i, 0)),
            pl.BlockSpec((_BR4, _DN), lambda i: (i, 0)),
            pl.BlockSpec((_BR4, _DE), lambda i: (i, 0)),
        ],
        out_shape=[
            jax.ShapeDtypeStruct((_N, _DN), jnp.float32),
            jax.ShapeDtypeStruct((_N, _DN), jnp.float32),
            jax.ShapeDtypeStruct((_N, _DE), jnp.float32),
        ],
    )(node, nm, am, deg, gb, We, Wn, be_row, bn_row)


def _k3_body(nblk, x_ref, ids_ref, ev_ref, wih_ref, whh_ref, bih_ref, bhh_ref,
             out_ref, h_ref, c_ref, qp_ref, rp_ref, m_ref, s_ref, r_ref):
    f32 = jnp.float32
    step = pl.program_id(0)
    blk = pl.program_id(1)
    NEG = -1e30

    @pl.when(jnp.logical_and(step == 0, blk == 0))
    def _init():
        h_ref[...] = jnp.zeros_like(h_ref)
        c_ref[...] = jnp.zeros_like(c_ref)
        qp_ref[...] = jnp.zeros_like(qp_ref)
        rp_ref[...] = jnp.zeros_like(rp_ref)

    @pl.when(blk == 0)
    def _lstm():
        q_star = jnp.concatenate([qp_ref[...], rp_ref[...]], axis=1)
        gates = (jnp.dot(q_star, wih_ref[...], preferred_element_type=f32)
                 + bih_ref[...]
                 + jnp.dot(h_ref[...], whh_ref[...], preferred_element_type=f32)
                 + bhh_ref[...])
        ig = jax.nn.sigmoid(gates[:, 0:128])
        fg = jax.nn.sigmoid(gates[:, 128:256])
        gg = jnp.tanh(gates[:, 256:384])
        og = jax.nn.sigmoid(gates[:, 384:512])
        c = fg * c_ref[...] + ig * gg
        h = og * jnp.tanh(c)
        c_ref[...] = c
        h_ref[...] = h
        m_ref[...] = jnp.full_like(m_ref, NEG)
        s_ref[...] = jnp.zeros_like(s_ref)
        r_ref[...] = jnp.zeros_like(r_ref)

    x = x_ref[...]
    ids = ids_ref[...]
    oh = (ids == jax.lax.broadcasted_iota(jnp.int32, (1, 64), 1)).astype(f32)
    q = h_ref[...]
    qg = jnp.dot(oh, q, preferred_element_type=f32)
    e = jnp.sum(x * qg, axis=1, keepdims=True)
    e = jnp.where(ev_ref[...] > 0.0, e, NEG)
    bm = jnp.max(jnp.where(oh > 0.0, e, NEG), axis=0, keepdims=True)
    new_m = jnp.maximum(m_ref[...], bm)
    scale = jnp.exp(m_ref[...] - new_m)
    me = jnp.sum(oh * new_m, axis=1, keepdims=True)
    a = jnp.exp(e - me) * ev_ref[...]
    w = oh * a
    m_ref[...] = new_m
    s_ref[...] = s_ref[...] * scale + jnp.sum(w, axis=0, keepdims=True)
    contrib = jax.lax.dot_general(x, w, (((0,), (0,)), ((), ())),
                                  preferred_element_type=f32)
    r_ref[...] = r_ref[...] * scale + contrib

    @pl.when(blk == nblk - 1)
    def _fin():
        s = s_ref[...]
        rfin = jnp.where(s > 0.0, r_ref[...] / s, 0.0)
        rp_ref[...] = rfin.T
        qp_ref[...] = h_ref[...]
        out_ref[:, 0:128] = h_ref[...]
        out_ref[:, 128:256] = rfin.T


def _s2s_fused(p, x, ids, evf, be):
    import functools
    from jax.experimental.pallas import tpu as pltpu
    n = x.shape[0]
    nblk = n // be
    body = functools.partial(_k3_body, nblk)
    f32 = jnp.float32
    return pl.pallas_call(
        body,
        grid=(_STEPS, nblk),
        in_specs=[
            pl.BlockSpec((be, _DN), lambda s, b: (b, 0)),
            pl.BlockSpec((be, 1), lambda s, b: (b, 0)),
            pl.BlockSpec((be, 1), lambda s, b: (b, 0)),
            pl.BlockSpec((256, 512), lambda s, b: (0, 0)),
            pl.BlockSpec((128, 512), lambda s, b: (0, 0)),
            pl.BlockSpec((1, 512), lambda s, b: (0, 0)),
            pl.BlockSpec((1, 512), lambda s, b: (0, 0)),
        ],
        out_specs=pl.BlockSpec((64, 256), lambda s, b: (0, 0)),
        out_shape=jax.ShapeDtypeStruct((64, 256), f32),
        scratch_shapes=[
            pltpu.VMEM((64, 128), f32),
            pltpu.VMEM((64, 128), f32),
            pltpu.VMEM((64, 128), f32),
            pltpu.VMEM((64, 128), f32),
            pltpu.VMEM((1, 64), f32),
            pltpu.VMEM((1, 64), f32),
            pltpu.VMEM((128, 64), f32),
        ],
    )(x, ids[:, None], evf[:, None], p["W_ih"], p["W_hh"],
      p["b_ih"][None, :], p["b_hh"][None, :])


def _s2s(p, x, ids, num, w):
    d = x.shape[1]
    h = jnp.zeros((num, d), x.dtype)
    c = jnp.zeros((num, d), x.dtype)
    q_star = jnp.zeros((num, 2 * d), x.dtype)
    for _ in range(_STEPS):
        gates = q_star @ p["W_ih"] + p["b_ih"] + h @ p["W_hh"] + p["b_hh"]
        i, f, g, o = jnp.split(gates, 4, axis=1)
        c = jax.nn.sigmoid(f) * c + jax.nn.sigmoid(i) * jnp.tanh(g)
        h = jax.nn.sigmoid(o) * jnp.tanh(c)
        q = h
        e = jnp.sum(x * q[ids], axis=-1)
        e = jnp.where(w, e, -jnp.inf)
        m = jax.ops.segment_max(e, ids, num_segments=num)
        a = jnp.where(w, jnp.exp(e - m[ids]), 0.0)
        s = jax.ops.segment_sum(a, ids, num_segments=num)
        a = jnp.where(w, a / s[ids], 0.0)
        r = jax.ops.segment_sum(a[:, None] * x, ids, num_segments=num)
        q_star = jnp.concatenate([q, r], axis=1)
    return q_star


def kernel(positions, atomic_numbers, batch, global_features, params):
    gn = params["gn"]
    We, be = _collapse(gn["edge"], gn["edge_dense"])
    Wn, bn = _collapse(gn["node"], gn["node_dense"])
    Wg, bg = _collapse(gn["glob"], gn["global_dense"])
    Wf, bf = _collapse([params["dense1"], params["dense2"]], params["out"])

    sq = jnp.sum(positions * positions, axis=1)
    d2 = sq[:, None] + sq[None, :] - 2.0 * (positions @ positions.T)
    mask = (d2 < _CUTOFF * _CUTOFF) & (~jnp.eye(_N, dtype=bool))
    src, dst = jnp.nonzero(mask, size=_EC, fill_value=0)
    n_edges = jnp.sum(mask.astype(jnp.int32))
    ev = jnp.arange(_EC, dtype=jnp.int32) < n_edges
    src = src.astype(jnp.int32)
    dst = dst.astype(jnp.int32)

    node = params["embedding"][atomic_numbers]
    glob = global_features
    gb = glob[batch]

    F = jnp.concatenate(
        [node, jnp.ones((_N, 1), jnp.float32), jnp.zeros((_N, 127), jnp.float32)],
        axis=1)
    ps = _pair_sums(mask.astype(jnp.int8), F)
    nm = ps[:, :_DN]
    deg = ps[:, _DN:_DN + 1]

    offset = jnp.linspace(0.0, _CUTOFF, _DE)
    step = _CUTOFF / (_DE - 1)
    coeff = -0.5 / (step * step)
    A = We[:128]
    B = We[128:256]
    CpI = We[256:384] + jnp.eye(_DE, dtype=jnp.float32)
    D = We[384:]
    na = node @ (A + B)
    gDbe = glob @ D + be
    dist = jnp.linalg.norm(positions[src] - positions[dst], axis=-1)
    nasum = na[src] + na[dst]
    gterm = 0.5 * (gDbe[batch[src]] + gDbe[batch[dst]])
    amask, xe = _edge_kernel(coeff, dist[:, None], ev.astype(jnp.float32)[:, None],
                             nasum, gterm, offset[None, :], CpI)

    am = jax.ops.segment_sum(amask, src, num_segments=_N)

    nupd, nnew, P = _node_kernel(node, nm, am, deg, gb, We, Wn,
                                 be[None, :], bn[None, :])

    S = jax.ops.segment_sum(P, batch, num_segments=_G)
    Dg = jax.ops.segment_sum(deg[:, 0], batch, num_segments=_G)
    has_e = (Dg > 0.0).astype(jnp.float32)[:, None]
    egmean = S / jnp.maximum(Dg, 1.0)[:, None] + has_e * (glob @ D)
    nsum = jax.ops.segment_sum(nupd, batch, num_segments=_G)
    cnt = jax.ops.segment_sum(jnp.ones((_N,), jnp.float32), batch, num_segments=_G)
    ngmean = nsum / jnp.maximum(cnt, 1.0)[:, None]
    gin = jnp.concatenate([egmean, ngmean, glob], axis=1)
    gnew = gin @ Wg + bg + glob

    node_r = _s2s_fused(params["s2s_nodes"], nnew, batch,
                        jnp.ones((_N,), jnp.float32), 4096)
    edge_r = _s2s_fused(params["s2s_edges"], xe, batch[src],
                        ev.astype(jnp.float32), _BE)
    y = jnp.concatenate([node_r, edge_r, gnew], axis=1) @ Wf + bf
    return y


# speedup vs baseline: 7.4358x; 1.2693x over previous
import jax
import jax.numpy as jnp
from jax.experimental import pallas as pl

_N = 4096
_G = 64
_DN = 128
_DE = 128
_DG = 64
_CUTOFF = 0.098
_STEPS = 3
_EC = 98304

_BR1 = 256
_BE = 4096
_BR4 = 512


def _collapse(layers, dense):
    W = layers[0]["W"]
    b = layers[0]["b"]
    for lp in list(layers[1:]) + [dense]:
        b = b @ lp["W"] + lp["b"]
        W = W @ lp["W"]
    return W, b


def _k1_body(mask_ref, f_ref, out_ref):
    m = mask_ref[...].astype(jnp.float32)
    out_ref[...] = jnp.dot(m, f_ref[...], preferred_element_type=jnp.float32)


def _pair_sums(mask_i8, F):
    return pl.pallas_call(
        _k1_body,
        grid=(_N // _BR1,),
        in_specs=[
            pl.BlockSpec((_BR1, _N), lambda i: (i, 0)),
            pl.BlockSpec((_N, 256), lambda i: (0, 0)),
        ],
        out_specs=pl.BlockSpec((_BR1, 256), lambda i: (i, 0)),
        out_shape=jax.ShapeDtypeStruct((_N, 256), jnp.float32),
    )(mask_i8, F)


def _k2_body(coeff, dist_ref, ev_ref, nasum_ref, bs_ref, bd_ref, gdbe_ref,
             off_ref, cpi_ref, amask_ref, xe_ref):
    f32 = jnp.float32
    d = dist_ref[...]
    a = jnp.exp(coeff * (d - off_ref[...]) ** 2)
    amask_ref[...] = a * ev_ref[...]
    giota = jax.lax.broadcasted_iota(jnp.int32, (1, 64), 1)
    ohg = 0.5 * ((bs_ref[...] == giota).astype(f32)
                 + (bd_ref[...] == giota).astype(f32))
    gterm = jnp.dot(ohg, gdbe_ref[...], preferred_element_type=f32)
    xe_ref[...] = (0.5 * nasum_ref[...]
                   + jnp.dot(a, cpi_ref[...], preferred_element_type=f32)
                   + gterm)


def _edge_kernel(coeff, dist, ev_f, nasum, bs, bd, gDbe, off_row, cpi):
    import functools
    body = functools.partial(_k2_body, coeff)
    return pl.pallas_call(
        body,
        grid=(_EC // _BE,),
        in_specs=[
            pl.BlockSpec((_BE, 1), lambda i: (i, 0)),
            pl.BlockSpec((_BE, 1), lambda i: (i, 0)),
            pl.BlockSpec((_BE, _DE), lambda i: (i, 0)),
            pl.BlockSpec((_BE, 1), lambda i: (i, 0)),
            pl.BlockSpec((_BE, 1), lambda i: (i, 0)),
            pl.BlockSpec((64, _DE), lambda i: (0, 0)),
            pl.BlockSpec((1, _DE), lambda i: (0, 0)),
            pl.BlockSpec((_DE, _DE), lambda i: (0, 0)),
        ],
        out_specs=[
            pl.BlockSpec((_BE, _DE), lambda i: (i, 0)),
            pl.BlockSpec((_BE, _DE), lambda i: (i, 0)),
        ],
        out_shape=[
            jax.ShapeDtypeStruct((_EC, _DE), jnp.float32),
            jax.ShapeDtypeStruct((_EC, _DE), jnp.float32),
        ],
    )(dist, ev_f, nasum, bs[:, None], bd[:, None], gDbe, off_row, cpi)


def _k4_body(node_ref, nm_ref, am_ref, deg_ref, gb_ref, we_ref, wn_ref,
             be_ref, bn_ref, nupd_ref, nnew_ref, p_ref):
    node = node_ref[...]
    deg = deg_ref[...]
    invd = 1.0 / jnp.maximum(deg, 1.0)
    has = (deg > 0.0).astype(jnp.float32)
    A = we_ref[0:128, :]
    B = we_ref[128:256, :]
    C = we_ref[256:384, :]
    D = we_ref[384:448, :]
    f32 = jnp.float32
    nA = jnp.dot(node, A, preferred_element_type=f32)
    nmB = jnp.dot(nm_ref[...], B, preferred_element_type=f32)
    amC = jnp.dot(am_ref[...], C, preferred_element_type=f32)
    gD = jnp.dot(gb_ref[...], D, preferred_element_type=f32)
    be = be_ref[...]
    emean = has * (nA + invd * (nmB + amC) + gD + be)
    p_ref[...] = deg * nA + nmB + amC + deg * be
    WnN = wn_ref[0:128, :]
    WnE = wn_ref[128:256, :]
    WnG = wn_ref[256:320, :]
    nupd = (jnp.dot(node, WnN, preferred_element_type=f32)
            + jnp.dot(emean, WnE, preferred_element_type=f32)
            + jnp.dot(gb_ref[...], WnG, preferred_element_type=f32)
            + bn_ref[...])
    nupd_ref[...] = nupd
    nnew_ref[...] = nupd + node


def _node_kernel(node, nm, am, deg, gb, We, Wn, be_row, bn_row):
    return pl.pallas_call(
        _k4_body,
        grid=(_N // _BR4,),
        in_specs=[
            pl.BlockSpec((_BR4, _DN), lambda i: (i, 0)),
            pl.BlockSpec((_BR4, _DN), lambda i: (i, 0)),
            pl.BlockSpec((_BR4, _DE), lambda i: (i, 0)),
            pl.BlockSpec((_BR4, 1), lambda i: (i, 0)),
            pl.BlockSpec((_BR4, _DG), lambda i: (i, 0)),
            pl.BlockSpec((448, 128), lambda i: (0, 0)),
            pl.BlockSpec((320, 128), lambda i: (0, 0)),
            pl.BlockSpec((1, 128), lambda i: (0, 0)),
            pl.BlockSpec((1, 128), lambda i: (0, 0)),
        ],
        out_specs=[
            pl.BlockSpec((_BR4, _DN), lambda i: (i, 0)),
            pl.BlockSpec((_BR4, _DN), lambda i: (i, 0)),
            pl.BlockSpec((_BR4, _DE), lambda i: (i, 0)),
        ],
        out_shape=[
            jax.ShapeDtypeStruct((_N, _DN), jnp.float32),
            jax.ShapeDtypeStruct((_N, _DN), jnp.float32),
            jax.ShapeDtypeStruct((_N, _DE), jnp.float32),
        ],
    )(node, nm, am, deg, gb, We, Wn, be_row, bn_row)


def _k3_body(nblk, x_ref, ids_ref, ev_ref, wih_ref, whh_ref, bih_ref, bhh_ref,
             out_ref, h_ref, c_ref, qp_ref, rp_ref, m_ref, s_ref, r_ref):
    f32 = jnp.float32
    step = pl.program_id(0)
    blk = pl.program_id(1)
    NEG = -1e30

    @pl.when(jnp.logical_and(step == 0, blk == 0))
    def _init():
        h_ref[...] = jnp.zeros_like(h_ref)
        c_ref[...] = jnp.zeros_like(c_ref)
        qp_ref[...] = jnp.zeros_like(qp_ref)
        rp_ref[...] = jnp.zeros_like(rp_ref)

    @pl.when(blk == 0)
    def _lstm():
        q_star = jnp.concatenate([qp_ref[...], rp_ref[...]], axis=1)
        gates = (jnp.dot(q_star, wih_ref[...], preferred_element_type=f32)
                 + bih_ref[...]
                 + jnp.dot(h_ref[...], whh_ref[...], preferred_element_type=f32)
                 + bhh_ref[...])
        ig = jax.nn.sigmoid(gates[:, 0:128])
        fg = jax.nn.sigmoid(gates[:, 128:256])
        gg = jnp.tanh(gates[:, 256:384])
        og = jax.nn.sigmoid(gates[:, 384:512])
        c = fg * c_ref[...] + ig * gg
        h = og * jnp.tanh(c)
        c_ref[...] = c
        h_ref[...] = h
        m_ref[...] = jnp.full_like(m_ref, NEG)
        s_ref[...] = jnp.zeros_like(s_ref)
        r_ref[...] = jnp.zeros_like(r_ref)

    x = x_ref[...]
    ids = ids_ref[...]
    oh = (ids == jax.lax.broadcasted_iota(jnp.int32, (1, 64), 1)).astype(f32)
    q = h_ref[...]
    qg = jnp.dot(oh, q, preferred_element_type=f32)
    e = jnp.sum(x * qg, axis=1, keepdims=True)
    e = jnp.where(ev_ref[...] > 0.0, e, NEG)
    bm = jnp.max(jnp.where(oh > 0.0, e, NEG), axis=0, keepdims=True)
    new_m = jnp.maximum(m_ref[...], bm)
    scale = jnp.exp(m_ref[...] - new_m)
    me = jnp.sum(oh * new_m, axis=1, keepdims=True)
    a = jnp.exp(e - me) * ev_ref[...]
    w = oh * a
    m_ref[...] = new_m
    s_ref[...] = s_ref[...] * scale + jnp.sum(w, axis=0, keepdims=True)
    contrib = jax.lax.dot_general(x, w, (((0,), (0,)), ((), ())),
                                  preferred_element_type=f32)
    r_ref[...] = r_ref[...] * scale + contrib

    @pl.when(blk == nblk - 1)
    def _fin():
        s = s_ref[...]
        rfin = jnp.where(s > 0.0, r_ref[...] / s, 0.0)
        rp_ref[...] = rfin.T
        qp_ref[...] = h_ref[...]
        out_ref[:, 0:128] = h_ref[...]
        out_ref[:, 128:256] = rfin.T


def _s2s_fused(p, x, ids, evf, be):
    import functools
    from jax.experimental.pallas import tpu as pltpu
    n = x.shape[0]
    nblk = n // be
    body = functools.partial(_k3_body, nblk)
    f32 = jnp.float32
    return pl.pallas_call(
        body,
        grid=(_STEPS, nblk),
        in_specs=[
            pl.BlockSpec((be, _DN), lambda s, b: (b, 0)),
            pl.BlockSpec((be, 1), lambda s, b: (b, 0)),
            pl.BlockSpec((be, 1), lambda s, b: (b, 0)),
            pl.BlockSpec((256, 512), lambda s, b: (0, 0)),
            pl.BlockSpec((128, 512), lambda s, b: (0, 0)),
            pl.BlockSpec((1, 512), lambda s, b: (0, 0)),
            pl.BlockSpec((1, 512), lambda s, b: (0, 0)),
        ],
        out_specs=pl.BlockSpec((64, 256), lambda s, b: (0, 0)),
        out_shape=jax.ShapeDtypeStruct((64, 256), f32),
        scratch_shapes=[
            pltpu.VMEM((64, 128), f32),
            pltpu.VMEM((64, 128), f32),
            pltpu.VMEM((64, 128), f32),
            pltpu.VMEM((64, 128), f32),
            pltpu.VMEM((1, 64), f32),
            pltpu.VMEM((1, 64), f32),
            pltpu.VMEM((128, 64), f32),
        ],
    )(x, ids[:, None], evf[:, None], p["W_ih"], p["W_hh"],
      p["b_ih"][None, :], p["b_hh"][None, :])


def _s2s(p, x, ids, num, w):
    d = x.shape[1]
    h = jnp.zeros((num, d), x.dtype)
    c = jnp.zeros((num, d), x.dtype)
    q_star = jnp.zeros((num, 2 * d), x.dtype)
    for _ in range(_STEPS):
        gates = q_star @ p["W_ih"] + p["b_ih"] + h @ p["W_hh"] + p["b_hh"]
        i, f, g, o = jnp.split(gates, 4, axis=1)
        c = jax.nn.sigmoid(f) * c + jax.nn.sigmoid(i) * jnp.tanh(g)
        h = jax.nn.sigmoid(o) * jnp.tanh(c)
        q = h
        e = jnp.sum(x * q[ids], axis=-1)
        e = jnp.where(w, e, -jnp.inf)
        m = jax.ops.segment_max(e, ids, num_segments=num)
        a = jnp.where(w, jnp.exp(e - m[ids]), 0.0)
        s = jax.ops.segment_sum(a, ids, num_segments=num)
        a = jnp.where(w, a / s[ids], 0.0)
        r = jax.ops.segment_sum(a[:, None] * x, ids, num_segments=num)
        q_star = jnp.concatenate([q, r], axis=1)
    return q_star


def kernel(positions, atomic_numbers, batch, global_features, params):
    gn = params["gn"]
    We, be = _collapse(gn["edge"], gn["edge_dense"])
    Wn, bn = _collapse(gn["node"], gn["node_dense"])
    Wg, bg = _collapse(gn["glob"], gn["global_dense"])
    Wf, bf = _collapse([params["dense1"], params["dense2"]], params["out"])

    sq = jnp.sum(positions * positions, axis=1)
    d2 = sq[:, None] + sq[None, :] - 2.0 * (positions @ positions.T)
    mask = (d2 < _CUTOFF * _CUTOFF) & (~jnp.eye(_N, dtype=bool))
    src, dst = jnp.nonzero(mask, size=_EC, fill_value=0)
    n_edges = jnp.sum(mask.astype(jnp.int32))
    ev = jnp.arange(_EC, dtype=jnp.int32) < n_edges
    src = src.astype(jnp.int32)
    dst = dst.astype(jnp.int32)

    node = params["embedding"][atomic_numbers]
    glob = global_features
    gb = glob[batch]

    F = jnp.concatenate(
        [node, jnp.ones((_N, 1), jnp.float32), jnp.zeros((_N, 127), jnp.float32)],
        axis=1)
    ps = _pair_sums(mask.astype(jnp.int8), F)
    nm = ps[:, :_DN]
    deg = ps[:, _DN:_DN + 1]

    offset = jnp.linspace(0.0, _CUTOFF, _DE)
    step = _CUTOFF / (_DE - 1)
    coeff = -0.5 / (step * step)
    A = We[:128]
    B = We[128:256]
    CpI = We[256:384] + jnp.eye(_DE, dtype=jnp.float32)
    D = We[384:]
    na = node @ (A + B)
    gDbe = glob @ D + be
    dist = jnp.linalg.norm(positions[src] - positions[dst], axis=-1)
    nasum = na[src] + na[dst]
    amask, xe = _edge_kernel(coeff, dist[:, None], ev.astype(jnp.float32)[:, None],
                             nasum, batch[src], batch[dst], gDbe,
                             offset[None, :], CpI)

    am = jax.ops.segment_sum(amask, src, num_segments=_N)

    nupd, nnew, P = _node_kernel(node, nm, am, deg, gb, We, Wn,
                                 be[None, :], bn[None, :])

    S = jax.ops.segment_sum(P, batch, num_segments=_G)
    Dg = jax.ops.segment_sum(deg[:, 0], batch, num_segments=_G)
    has_e = (Dg > 0.0).astype(jnp.float32)[:, None]
    egmean = S / jnp.maximum(Dg, 1.0)[:, None] + has_e * (glob @ D)
    nsum = jax.ops.segment_sum(nupd, batch, num_segments=_G)
    cnt = jax.ops.segment_sum(jnp.ones((_N,), jnp.float32), batch, num_segments=_G)
    ngmean = nsum / jnp.maximum(cnt, 1.0)[:, None]
    gin = jnp.concatenate([egmean, ngmean, glob], axis=1)
    gnew = gin @ Wg + bg + glob

    node_r = _s2s_fused(params["s2s_nodes"], nnew, batch,
                        jnp.ones((_N,), jnp.float32), 4096)
    edge_r = _s2s_fused(params["s2s_edges"], xe, batch[src],
                        ev.astype(jnp.float32), _BE)
    y = jnp.concatenate([node_r, edge_r, gnew], axis=1) @ Wf + bf
    return y
